# Initial kernel scaffold; baseline (speedup 1.0000x reference)
import jax
import jax.numpy as jnp
from jax.experimental import pallas as pl


def kernel(xyz):
    raise NotImplementedError("write your pallas kernel here")


# speedup vs baseline: 1.6153x; 1.6153x over previous
import functools

import jax
import jax.numpy as jnp
from jax.experimental import pallas as pl
from jax.experimental.pallas import tpu as pltpu

B, N, G, K = 16, 8192, 128, 32


def _fps_body(xt_ref, centers_ref, dist_ref):
    X = xt_ref[0]
    Y = xt_ref[1]
    Z = xt_ref[2]
    lane = jax.lax.broadcasted_iota(jnp.int32, (B, N), 1)
    dist_ref[...] = jnp.full((B, N), 1e10, dtype=jnp.float32)

    def step(i, far):
        mask = lane == far
        cx = jnp.sum(jnp.where(mask, X, 0.0), axis=1, keepdims=True)
        cy = jnp.sum(jnp.where(mask, Y, 0.0), axis=1, keepdims=True)
        cz = jnp.sum(jnp.where(mask, Z, 0.0), axis=1, keepdims=True)
        centers_ref[i, :, :] = jnp.concatenate([cx, cy, cz], axis=1)
        dx = X - cx
        dy = Y - cy
        dz = Z - cz
        d = dx * dx + dy * dy + dz * dz
        nd = jnp.minimum(dist_ref[...], d)
        dist_ref[...] = nd
        m = jnp.max(nd, axis=1, keepdims=True)
        far2 = jnp.min(jnp.where(nd == m, lane, N), axis=1, keepdims=True)
        return far2

    jax.lax.fori_loop(0, G, step, jnp.zeros((B, 1), jnp.int32))


@functools.partial(jax.jit, static_argnames=("interpret",))
def _fps_centers(xyz, interpret=False):
    xt = jnp.transpose(xyz, (2, 0, 1))
    centers_gb3 = pl.pallas_call(
        _fps_body,
        out_shape=jax.ShapeDtypeStruct((G, B, 3), jnp.float32),
        scratch_shapes=[pltpu.VMEM((B, N), jnp.float32)],
        interpret=interpret,
    )(xt)
    return jnp.transpose(centers_gb3, (1, 0, 2))


def kernel(xyz):
    center = _fps_centers(xyz)
    dist = jnp.linalg.norm(center[:, :, None, :] - xyz[:, None, :, :], axis=-1)
    _, idx = jax.lax.top_k(-dist, K)
    flat = idx.reshape(B, G * K)
    patch = jnp.take_along_axis(xyz, flat[:, :, None], axis=1).reshape(B, G, K, 3)
    patch = patch - center[:, :, None, :]
    return (patch, center)


# speedup vs baseline: 9.3932x; 5.8151x over previous
import functools

import jax
import jax.numpy as jnp
from jax import lax
from jax.experimental import pallas as pl
from jax.experimental.pallas import tpu as pltpu
from jax.experimental.pallas import tpu_sc as plsc

B, N, G, K = 16, 8192, 128, 32
L = 16
GH = G // 2
NV = N // L
CHUNK = 16
NCHUNK = NV // CHUNK
INF = 3.0e38


def _fps_body(xt_ref, centers_ref, dist_ref):
    X = xt_ref[0]
    Y = xt_ref[1]
    Z = xt_ref[2]
    lane = jax.lax.broadcasted_iota(jnp.int32, (B, N), 1)
    dist_ref[...] = jnp.full((B, N), 1e10, dtype=jnp.float32)

    def step(i, far):
        mask = lane == far
        cx = jnp.sum(jnp.where(mask, X, 0.0), axis=1, keepdims=True)
        cy = jnp.sum(jnp.where(mask, Y, 0.0), axis=1, keepdims=True)
        cz = jnp.sum(jnp.where(mask, Z, 0.0), axis=1, keepdims=True)
        centers_ref[i, :, :] = jnp.concatenate([cx, cy, cz], axis=1)
        dx = X - cx
        dy = Y - cy
        dz = Z - cz
        d = dx * dx + dy * dy + dz * dz
        nd = jnp.minimum(dist_ref[...], d)
        dist_ref[...] = nd
        m = jnp.max(nd, axis=1, keepdims=True)
        far2 = jnp.min(jnp.where(nd == m, lane, N), axis=1, keepdims=True)
        return far2

    jax.lax.fori_loop(0, G, step, jnp.zeros((B, 1), jnp.int32))


def _fps_centers(xyz):
    xt = jnp.transpose(xyz, (2, 0, 1))
    centers_gb3 = pl.pallas_call(
        _fps_body,
        out_shape=jax.ShapeDtypeStruct((G, B, 3), jnp.float32),
        scratch_shapes=[pltpu.VMEM((B, N), jnp.float32)],
    )(xt)
    return centers_gb3


def _knn_body(xt_hbm, ct_hbm, out_hbm, xv, yv, zv, cv, db, cd, cix, pbuf):
    c_ax = lax.axis_index("c")
    s_ax = lax.axis_index("s")
    b = s_ax
    gh = c_ax

    pltpu.sync_copy(xt_hbm.at[pl.ds(b * (3 * N), N)], xv)
    pltpu.sync_copy(xt_hbm.at[pl.ds(b * (3 * N) + N, N)], yv)
    pltpu.sync_copy(xt_hbm.at[pl.ds(b * (3 * N) + 2 * N, N)], zv)
    pltpu.sync_copy(ct_hbm.at[pl.ds(b * (3 * G), 3 * G)], cv)

    lane = lax.broadcasted_iota(jnp.int32, (L,), 0)
    inf_v = jnp.full((L,), INF, dtype=jnp.float32)
    bigi_v = jnp.full((L,), N, dtype=jnp.int32)

    def per_group(g, _):
        gg = gh * GH + g
        ggv = jnp.full((L,), 0, jnp.int32) + gg
        cgx = plsc.load_gather(cv, [ggv])
        cgy = plsc.load_gather(cv, [ggv + G])
        cgz = plsc.load_gather(cv, [ggv + 2 * G])

        def chunk1(c, carry):
            m1, m2 = carry
            base = c * (CHUNK * L)
            for v in range(CHUNK):
                sl = pl.ds(base + v * L, L)
                dx = xv[sl] - cgx
                dy = yv[sl] - cgy
                dz = zv[sl] - cgz
                d = dx * dx + dy * dy + dz * dz
                db[sl] = d
                m2 = jnp.minimum(m2, jnp.maximum(m1, d))
                m1 = jnp.minimum(m1, d)
            return m1, m2

        m1, m2 = lax.fori_loop(0, NCHUNK, chunk1, (inf_v, inf_v))
        tau = jnp.max(m2)

        def chunk2(c, cur):
            base = c * (CHUNK * L)
            for v in range(CHUNK):
                sl = pl.ds(base + v * L, L)
                d = db[sl]
                msk = d <= tau
                mi32 = msk.astype(jnp.int32)
                pos = cur + plsc.cumsum(mi32) - 1
                plsc.store_scatter(cd, [pos], d, mask=msk)
                nvec = lane + (base + v * L)
                plsc.store_scatter(cix, [pos], nvec, mask=msk)
                cur = cur + plsc.all_reduce_population_count(msk)
            return cur

        cur_v = lax.fori_loop(0, NCHUNK, chunk2, jnp.zeros((L,), jnp.int32))
        cursor = jnp.max(cur_v)
        plsc.store_scatter(cd, [cursor + lane], inf_v)

        nvc = (cursor + (L - 1)) // L

        prev = jnp.int32(-1)
        for t in range(K // L):
            accv = jnp.zeros((L,), jnp.int32)
            for jj in range(L):
                def scan(v, carry, _prev=prev):
                    m, mi = carry
                    sl = pl.ds(v * L, L)
                    dv = cd[sl]
                    iv = cix[sl]
                    dv2 = jnp.where(iv == _prev, INF, dv)
                    cd[sl] = dv2
                    upd = (dv2 < m) | ((dv2 == m) & (iv < mi))
                    m = jnp.where(upd, dv2, m)
                    mi = jnp.where(upd, iv, mi)
                    return m, mi

                m, mi = lax.fori_loop(0, nvc, scan, (inf_v, bigi_v))
                dmin = jnp.min(m)
                imin = jnp.min(jnp.where(m == dmin, mi, N))
                accv = jnp.where(lane == jj, imin, accv)
                prev = imin

            px = plsc.load_gather(xv, [accv]) - cgx
            py = plsc.load_gather(yv, [accv]) - cgy
            pz = plsc.load_gather(zv, [accv]) - cgz
            pos = (g * K + t * L) * 3 + lane * 3
            plsc.store_scatter(pbuf, [pos], px)
            plsc.store_scatter(pbuf, [pos + 1], py)
            plsc.store_scatter(pbuf, [pos + 2], pz)
        return 0

    lax.fori_loop(0, GH, per_group, 0)
    pltpu.sync_copy(pbuf, out_hbm.at[pl.ds((b * 2 + gh) * (GH * K * 3), GH * K * 3)])


def _knn_patch_sc(xyz, centers_gb3):
    xt = jnp.transpose(xyz, (0, 2, 1)).reshape(B * 3 * N)
    ct = jnp.transpose(centers_gb3, (1, 2, 0)).reshape(B * 3 * G)
    mesh = plsc.VectorSubcoreMesh(core_axis_name="c", subcore_axis_name="s")
    out = pl.kernel(
        _knn_body,
        out_type=jax.ShapeDtypeStruct((B * 2 * GH * K * 3,), jnp.float32),
        mesh=mesh,
        compiler_params=pltpu.CompilerParams(needs_layout_passes=False),
        scratch_types=[
            pltpu.VMEM((N,), jnp.float32),
            pltpu.VMEM((N,), jnp.float32),
            pltpu.VMEM((N,), jnp.float32),
            pltpu.VMEM((3 * G,), jnp.float32),
            pltpu.VMEM((N,), jnp.float32),
            pltpu.VMEM((N + L,), jnp.float32),
            pltpu.VMEM((N + L,), jnp.int32),
            pltpu.VMEM((GH * K * 3,), jnp.float32),
        ],
    )(xt, ct)
    return out.reshape(B, G, K, 3)


def kernel(xyz):
    centers_gb3 = _fps_centers(xyz)
    center = jnp.transpose(centers_gb3, (1, 0, 2))
    patch = _knn_patch_sc(xyz, centers_gb3)
    return (patch, center)


# speedup vs baseline: 10.3765x; 1.1047x over previous
import functools

import jax
import jax.numpy as jnp
from jax import lax
from jax.experimental import pallas as pl
from jax.experimental.pallas import tpu as pltpu
from jax.experimental.pallas import tpu_sc as plsc

B, N, G, K = 16, 8192, 128, 32
L = 16
GH = G // 2
NV = N // L
CHUNK = 16
NCHUNK = NV // CHUNK
INF = 3.0e38


def _fps_body(xt_ref, centers_ref, dist_ref):
    X = xt_ref[0]
    Y = xt_ref[1]
    Z = xt_ref[2]
    lane = jax.lax.broadcasted_iota(jnp.int32, (B, N), 1)
    dist_ref[...] = jnp.full((B, N), 1e10, dtype=jnp.float32)

    def step(i, far):
        mask = lane == far
        cx = jnp.sum(jnp.where(mask, X, 0.0), axis=1, keepdims=True)
        cy = jnp.sum(jnp.where(mask, Y, 0.0), axis=1, keepdims=True)
        cz = jnp.sum(jnp.where(mask, Z, 0.0), axis=1, keepdims=True)
        centers_ref[i, :, :] = jnp.concatenate([cx, cy, cz], axis=1)
        dx = X - cx
        dy = Y - cy
        dz = Z - cz
        d = dx * dx + dy * dy + dz * dz
        nd = jnp.minimum(dist_ref[...], d)
        dist_ref[...] = nd
        m = jnp.max(nd, axis=1, keepdims=True)
        far2 = jnp.min(jnp.where(nd == m, lane, N), axis=1, keepdims=True)
        return far2

    jax.lax.fori_loop(0, G, step, jnp.zeros((B, 1), jnp.int32))


def _fps_centers(xyz):
    xt = jnp.transpose(xyz, (2, 0, 1))
    centers_gb3 = pl.pallas_call(
        _fps_body,
        out_shape=jax.ShapeDtypeStruct((G, B, 3), jnp.float32),
        scratch_shapes=[pltpu.VMEM((B, N), jnp.float32)],
    )(xt)
    return centers_gb3


def _knn_body(xt_hbm, ct_hbm, out_hbm, xv, yv, zv, cv, db, cd, cix, pbuf):
    c_ax = lax.axis_index("c")
    s_ax = lax.axis_index("s")
    b = s_ax
    gh = c_ax

    pltpu.sync_copy(xt_hbm.at[pl.ds(b * (3 * N), N)], xv)
    pltpu.sync_copy(xt_hbm.at[pl.ds(b * (3 * N) + N, N)], yv)
    pltpu.sync_copy(xt_hbm.at[pl.ds(b * (3 * N) + 2 * N, N)], zv)
    pltpu.sync_copy(ct_hbm.at[pl.ds(b * (3 * G), 3 * G)], cv)

    lane = lax.broadcasted_iota(jnp.int32, (L,), 0)
    inf_v = jnp.full((L,), INF, dtype=jnp.float32)
    bigi_v = jnp.full((L,), N, dtype=jnp.int32)

    def per_group(g, _):
        gg = gh * GH + g
        ggv = jnp.full((L,), 0, jnp.int32) + gg
        cgx = plsc.load_gather(cv, [ggv])
        cgy = plsc.load_gather(cv, [ggv + G])
        cgz = plsc.load_gather(cv, [ggv + 2 * G])

        def chunk1(c, carry):
            m1, m2 = carry
            base = c * (CHUNK * L)
            for v in range(CHUNK):
                sl = pl.ds(base + v * L, L)
                dx = xv[sl] - cgx
                dy = yv[sl] - cgy
                dz = zv[sl] - cgz
                d = dx * dx + dy * dy + dz * dz
                db[sl] = d
                m2 = jnp.minimum(m2, jnp.maximum(m1, d))
                m1 = jnp.minimum(m1, d)
            return m1, m2

        m1, m2 = lax.fori_loop(0, NCHUNK, chunk1, (inf_v, inf_v))
        tau = jnp.max(m2)

        def chunk2(c, cur):
            base = c * (CHUNK * L)
            for v in range(CHUNK):
                sl = pl.ds(base + v * L, L)
                d = db[sl]
                msk = d <= tau
                plsc.store_compressed(cd.at[pl.ds(cur, L)], d, mask=msk)
                nvec = lane + (base + v * L)
                plsc.store_compressed(cix.at[pl.ds(cur, L)], nvec, mask=msk)
                cur = cur + plsc.all_reduce_population_count(msk)[0]
            return cur

        cursor = lax.fori_loop(0, NCHUNK, chunk2, jnp.int32(0))
        plsc.store_scatter(cd, [cursor + lane], inf_v)

        nvc = (cursor + (L - 1)) // L

        for t in range(K // L):
            accv = jnp.zeros((L,), jnp.int32)
            for jj in range(L):
                def scan(v, carry):
                    m, mi, mp = carry
                    sl = pl.ds(v * L, L)
                    dv = cd[sl]
                    iv = cix[sl]
                    upd = (dv < m) | ((dv == m) & (iv < mi))
                    m = jnp.where(upd, dv, m)
                    mi = jnp.where(upd, iv, mi)
                    mp = jnp.where(upd, lane + v * L, mp)
                    return m, mi, mp

                m, mi, mp = lax.fori_loop(0, nvc, scan, (inf_v, bigi_v, bigi_v))
                dmin = jnp.min(m)
                win = m == dmin
                imin = jnp.min(jnp.where(win, mi, N))
                pmin = jnp.min(jnp.where(win & (mi == imin), mp, N + L))
                plsc.store_scatter(cd, [jnp.full((L,), 0, jnp.int32) + pmin],
                                   inf_v, mask=lane == 0)
                accv = jnp.where(lane == jj, imin, accv)

            px = plsc.load_gather(xv, [accv]) - cgx
            py = plsc.load_gather(yv, [accv]) - cgy
            pz = plsc.load_gather(zv, [accv]) - cgz
            pos = (g * K + t * L) * 3 + lane * 3
            plsc.store_scatter(pbuf, [pos], px)
            plsc.store_scatter(pbuf, [pos + 1], py)
            plsc.store_scatter(pbuf, [pos + 2], pz)
        return 0

    lax.fori_loop(0, GH, per_group, 0)
    pltpu.sync_copy(pbuf, out_hbm.at[pl.ds((b * 2 + gh) * (GH * K * 3), GH * K * 3)])


def _knn_patch_sc(xyz, centers_gb3):
    xt = jnp.transpose(xyz, (0, 2, 1)).reshape(B * 3 * N)
    ct = jnp.transpose(centers_gb3, (1, 2, 0)).reshape(B * 3 * G)
    mesh = plsc.VectorSubcoreMesh(core_axis_name="c", subcore_axis_name="s")
    out = pl.kernel(
        _knn_body,
        out_type=jax.ShapeDtypeStruct((B * 2 * GH * K * 3,), jnp.float32),
        mesh=mesh,
        compiler_params=pltpu.CompilerParams(needs_layout_passes=False),
        scratch_types=[
            pltpu.VMEM((N,), jnp.float32),
            pltpu.VMEM((N,), jnp.float32),
            pltpu.VMEM((N,), jnp.float32),
            pltpu.VMEM((3 * G,), jnp.float32),
            pltpu.VMEM((N,), jnp.float32),
            pltpu.VMEM((N + L,), jnp.float32),
            pltpu.VMEM((N + L,), jnp.int32),
            pltpu.VMEM((GH * K * 3,), jnp.float32),
        ],
    )(xt, ct)
    return out.reshape(B, G, K, 3)


def kernel(xyz):
    centers_gb3 = _fps_centers(xyz)
    center = jnp.transpose(centers_gb3, (1, 0, 2))
    patch = _knn_patch_sc(xyz, centers_gb3)
    return (patch, center)
